# baseline (speedup 1.0000x reference)
import functools

import jax
import jax.numpy as jnp
from jax import lax
from jax.experimental import pallas as pl
from jax.experimental.pallas import tpu as pltpu
from jax.experimental.pallas import tpu_sc as plsc

V = 10000
C = 4
F = 128
K = 32
EPS = 1e-4
NLANE = 16
NW = 32
NBASE = V // NW
NBIG = NBASE + 1
NREM = V % NW
NSTAGE = 320
TRIL = [(0, 0), (1, 0), (1, 1), (2, 0), (2, 1), (2, 2),
        (3, 0), (3, 1), (3, 2), (3, 3)]
NY = 15
COVW = F * len(TRIL)
OUTW = COVW + F * C
PASSES = [(0, 3), (3, 3), (6, 2)]


def _body(coords_hbm, feats_hbm, nidx_hbm, out_hbm,
          coords_v, idx_v, w_v, y_v, orow_v, gsems, osems):
    wid = lax.axis_index("s") * 2 + lax.axis_index("c")
    big = wid < NREM
    base = jnp.where(big, wid * NBIG, NREM * NBIG + (wid - NREM) * NBASE)
    count = jnp.where(big, NBIG, NBASE)
    copy_base = jnp.minimum((base // 8) * 8, V - NSTAGE)
    off = base - copy_base

    pltpu.sync_copy(coords_hbm, coords_v)
    pltpu.sync_copy(nidx_hbm.at[pl.ds(copy_base * K, NSTAGE * K)], idx_v)

    iota = lax.iota(jnp.int32, NLANE)

    def node_ids(n):
        row = off + n
        return [idx_v[pl.ds(row * K + h * NLANE, NLANE)] for h in (0, 1)]

    def fire_gather(n, buf):
        ids_ref = idx_v.at[pl.ds((off + n) * K, K)]
        pltpu.make_async_copy(
            feats_hbm.at[ids_ref], w_v.at[buf], gsems[buf]).start()

    def wait_gather(buf):
        pltpu.make_async_copy(
            feats_hbm.at[pl.ds(0, K)], w_v.at[buf], gsems[buf]).wait()

    def drain_out(buf):
        pltpu.make_async_copy(orow_v.at[pl.ds(buf * OUTW, OUTW)],
                              out_hbm.at[pl.ds(0, OUTW)],
                              osems[buf]).wait()

    def do_node(n, buf, first):
        ids = node_ids(n)
        for h in (0, 1):
            flat = ids[h] * C
            xs = [plsc.load_gather(coords_v, [flat + c]) for c in range(C)]
            rowstart = (iota + h * NLANE) * NLANE
            for c in range(C):
                plsc.store_scatter(y_v, [rowstart + (1 + c)], xs[c])
            for t, (r, c) in enumerate(TRIL):
                plsc.store_scatter(y_v, [rowstart + (5 + t)], xs[r] * xs[c])

        wait_gather(buf)
        if not first:
            drain_out(buf)

        accs = []
        for cs, nch in PASSES:
            def k_body(k, acc):
                yk = y_v[pl.ds(k * NLANE, NLANE)]
                ys = [yk[j] for j in range(1, NY)]
                new = list(acc)
                for ci in range(nch):
                    col = (cs + ci) * NLANE
                    wv = w_v[buf, k, pl.ds(col, NLANE)]
                    a = new[ci * NY:(ci + 1) * NY]
                    a = ([a[0] + wv]
                         + [a[j] + wv * ys[j - 1] for j in range(1, NY)])
                    new[ci * NY:(ci + 1) * NY] = a
                return tuple(new)

            zeros = tuple(jnp.zeros((NLANE,), jnp.float32)
                          for _ in range(nch * NY))
            accs.append((cs, nch,
                         lax.fori_loop(0, K, k_body, zeros, unroll=4)))

        @pl.when(n + 2 < count)
        def _():
            fire_gather(n + 2, buf)

        for cs, nch, acc in accs:
            for ci in range(nch):
                f0 = (cs + ci) * NLANE
                a = acc[ci * NY:(ci + 1) * NY]
                recip = 1.0 / (a[0] + EPS)
                m = [a[1 + c] * recip for c in range(C)]
                cov_base = buf * OUTW + (iota + f0) * len(TRIL)
                for t, (r, c) in enumerate(TRIL):
                    cov = a[5 + t] * recip - m[r] * m[c]
                    plsc.store_scatter(orow_v, [cov_base + t], cov)
                mean_base = buf * OUTW + COVW + (iota + f0) * C
                for c in range(C):
                    plsc.store_scatter(orow_v, [mean_base + c], m[c])

        pltpu.make_async_copy(orow_v.at[pl.ds(buf * OUTW, OUTW)],
                              out_hbm.at[pl.ds((base + n) * OUTW, OUTW)],
                              osems[buf]).start()

    fire_gather(0, 0)
    fire_gather(1, 1)

    def pair_body(i, carry):
        n0 = 2 * i

        @pl.when(n0 < count)
        def _():
            do_node(n0, 0, first=False)

        @pl.when(n0 + 1 < count)
        def _():
            do_node(n0 + 1, 1, first=False)

        return carry

    do_node(0, 0, first=True)
    do_node(1, 1, first=True)
    lax.fori_loop(1, (NBIG + 1) // 2, pair_body, 0)

    drain_out(0)
    drain_out(1)


@jax.jit
def kernel(coordinates, features, n_idxs):
    mesh = plsc.VectorSubcoreMesh(core_axis_name="c", subcore_axis_name="s")
    k = functools.partial(
        pl.kernel,
        out_type=jax.ShapeDtypeStruct((V * OUTW,), jnp.float32),
        mesh=mesh,
        compiler_params=pltpu.CompilerParams(needs_layout_passes=False),
        scratch_types=[
            pltpu.VMEM((V * C,), jnp.float32),
            pltpu.VMEM((NSTAGE * K,), jnp.int32),
            pltpu.VMEM((2, K, F), jnp.float32),
            pltpu.VMEM((K * NLANE,), jnp.float32),
            pltpu.VMEM((2 * OUTW,), jnp.float32),
            (pltpu.SemaphoreType.DMA, pltpu.SemaphoreType.DMA),
            (pltpu.SemaphoreType.DMA, pltpu.SemaphoreType.DMA),
        ],
    )(_body)
    return k(coordinates.reshape(-1), features,
             n_idxs.reshape(-1)).reshape(V, OUTW)


# speedup vs baseline: 1.7940x; 1.7940x over previous
import functools

import jax
import jax.numpy as jnp
from jax import lax
from jax.experimental import pallas as pl
from jax.experimental.pallas import tpu as pltpu
from jax.experimental.pallas import tpu_sc as plsc

V = 10000
C = 4
F = 128
K = 32
EPS = 1e-4
NLANE = 16
NW = 32
NBASE = V // NW
NBIG = NBASE + 1
NREM = V % NW
NSTAGE = 320
TRIL = [(0, 0), (1, 0), (1, 1), (2, 0), (2, 1), (2, 2),
        (3, 0), (3, 1), (3, 2), (3, 3)]
NY = 15
COVW = F * len(TRIL)
OUTW = COVW + F * C
PASSES = [(0, 3), (3, 3), (6, 2)]


def _body(coords_hbm, feats_hbm, nidx_hbm, out_hbm,
          coords_v, idx_v, w_v, y_v, orow_v, gsems, osems):
    wid = lax.axis_index("s") * 2 + lax.axis_index("c")
    big = wid < NREM
    base = jnp.where(big, wid * NBIG, NREM * NBIG + (wid - NREM) * NBASE)
    count = jnp.where(big, NBIG, NBASE)
    copy_base = jnp.minimum((base // 8) * 8, V - NSTAGE)
    off = base - copy_base

    pltpu.sync_copy(coords_hbm, coords_v)
    pltpu.sync_copy(nidx_hbm.at[pl.ds(copy_base * K, NSTAGE * K)], idx_v)

    iota = lax.iota(jnp.int32, NLANE)

    def node_ids(n):
        row = off + n
        return [idx_v[pl.ds(row * K + h * NLANE, NLANE)] for h in (0, 1)]

    def fire_gather(n, buf):
        ids_ref = idx_v.at[pl.ds((off + n) * K, K)]
        pltpu.make_async_copy(
            feats_hbm.at[ids_ref], w_v.at[buf], gsems[buf]).start()

    def wait_gather(buf):
        pltpu.make_async_copy(
            feats_hbm.at[pl.ds(0, K)], w_v.at[buf], gsems[buf]).wait()

    def drain_out(buf):
        pltpu.make_async_copy(orow_v.at[pl.ds(buf * OUTW, OUTW)],
                              out_hbm.at[pl.ds(0, OUTW)],
                              osems[buf]).wait()

    def do_node(n, buf):
        ids = node_ids(n)
        for h in (0, 1):
            flat = ids[h] * C
            xs = [plsc.load_gather(coords_v, [flat + c]) for c in range(C)]
            rowstart = (iota + h * NLANE) * NLANE
            for c in range(C):
                plsc.store_scatter(y_v, [rowstart + c], xs[c])

        wait_gather(buf)
        @pl.when(n >= 2)
        def _():
            drain_out(buf)

        accs = []
        for cs, nch in PASSES:
            def k_body(k, acc):
                xk = y_v[pl.ds(k * NLANE, NLANE)]
                xb = [xk[c] for c in range(C)]
                new = list(acc)
                for ci in range(nch):
                    col = (cs + ci) * NLANE
                    wv = w_v[buf, k, pl.ds(col, NLANE)]
                    a = list(new[ci * NY:(ci + 1) * NY])
                    p = [wv * xb[c] for c in range(C)]
                    a[0] = a[0] + wv
                    for c in range(C):
                        a[1 + c] = a[1 + c] + p[c]
                    for t, (r, c) in enumerate(TRIL):
                        a[5 + t] = a[5 + t] + p[c] * xb[r]
                    new[ci * NY:(ci + 1) * NY] = a
                return tuple(new)

            zeros = tuple(jnp.zeros((NLANE,), jnp.float32)
                          for _ in range(nch * NY))
            accs.append((cs, nch,
                         lax.fori_loop(0, K, k_body, zeros, unroll=2)))

        @pl.when(n + 2 < count)
        def _():
            fire_gather(n + 2, buf)

        for cs, nch, acc in accs:
            for ci in range(nch):
                f0 = (cs + ci) * NLANE
                a = acc[ci * NY:(ci + 1) * NY]
                recip = 1.0 / (a[0] + EPS)
                m = [a[1 + c] * recip for c in range(C)]
                cov_base = buf * OUTW + (iota + f0) * len(TRIL)
                for t, (r, c) in enumerate(TRIL):
                    cov = a[5 + t] * recip - m[r] * m[c]
                    plsc.store_scatter(orow_v, [cov_base + t], cov)
                mean_base = buf * OUTW + COVW + (iota + f0) * C
                for c in range(C):
                    plsc.store_scatter(orow_v, [mean_base + c], m[c])

        pltpu.make_async_copy(orow_v.at[pl.ds(buf * OUTW, OUTW)],
                              out_hbm.at[pl.ds((base + n) * OUTW, OUTW)],
                              osems[buf]).start()

    fire_gather(0, 0)
    fire_gather(1, 1)

    def pair_body(i, carry):
        n0 = 2 * i

        @pl.when(n0 < count)
        def _():
            do_node(n0, 0)

        @pl.when(n0 + 1 < count)
        def _():
            do_node(n0 + 1, 1)

        return carry

    lax.fori_loop(0, (NBIG + 1) // 2, pair_body, 0)

    drain_out(0)
    drain_out(1)


@jax.jit
def kernel(coordinates, features, n_idxs):
    mesh = plsc.VectorSubcoreMesh(core_axis_name="c", subcore_axis_name="s")
    k = functools.partial(
        pl.kernel,
        out_type=jax.ShapeDtypeStruct((V * OUTW,), jnp.float32),
        mesh=mesh,
        compiler_params=pltpu.CompilerParams(needs_layout_passes=False),
        scratch_types=[
            pltpu.VMEM((V * C,), jnp.float32),
            pltpu.VMEM((NSTAGE * K,), jnp.int32),
            pltpu.VMEM((2, K, F), jnp.float32),
            pltpu.VMEM((K * NLANE,), jnp.float32),
            pltpu.VMEM((2 * OUTW,), jnp.float32),
            (pltpu.SemaphoreType.DMA, pltpu.SemaphoreType.DMA),
            (pltpu.SemaphoreType.DMA, pltpu.SemaphoreType.DMA),
        ],
    )(_body)
    return k(coordinates.reshape(-1), features,
             n_idxs.reshape(-1)).reshape(V, OUTW)


# speedup vs baseline: 2.8024x; 1.5621x over previous
import functools

import jax
import jax.numpy as jnp
from jax import lax
from jax.experimental import pallas as pl
from jax.experimental.pallas import tpu as pltpu
from jax.experimental.pallas import tpu_sc as plsc

V = 10000
C = 4
F = 128
K = 32
EPS = 1e-4
NLANE = 16
NW = 32
NBASE = V // NW
NBIG = NBASE + 1
NREM = V % NW
NSTAGE = 320
TRIL = [(0, 0), (1, 0), (1, 1), (2, 0), (2, 1), (2, 2),
        (3, 0), (3, 1), (3, 2), (3, 3)]
NY = 15
COVW = F * len(TRIL)
OUTW = COVW + F * C
PASSES = [(0, 3), (3, 3), (6, 2)]


def _body(coords_hbm, feats_hbm, nidx_hbm, out_hbm,
          coords_v, idx_v, w_v, y_v, orow_v, gsems, osems):
    wid = lax.axis_index("s") * 2 + lax.axis_index("c")
    big = wid < NREM
    base = jnp.where(big, wid * NBIG, NREM * NBIG + (wid - NREM) * NBASE)
    count = jnp.where(big, NBIG, NBASE)
    copy_base = jnp.minimum((base // 8) * 8, V - NSTAGE)
    off = base - copy_base

    pltpu.sync_copy(coords_hbm, coords_v)
    pltpu.sync_copy(nidx_hbm.at[pl.ds(copy_base * K, NSTAGE * K)], idx_v)

    iota = lax.iota(jnp.int32, NLANE)

    def node_ids(n):
        row = off + n
        return [idx_v[pl.ds(row * K + h * NLANE, NLANE)] for h in (0, 1)]

    def fire_gather(n, buf):
        ids_ref = idx_v.at[pl.ds((off + n) * K, K)]
        pltpu.make_async_copy(
            feats_hbm.at[ids_ref], w_v.at[buf], gsems[buf]).start()

    def wait_gather(buf):
        pltpu.make_async_copy(
            feats_hbm.at[pl.ds(0, K)], w_v.at[buf], gsems[buf]).wait()

    def drain_out(buf):
        pltpu.make_async_copy(orow_v.at[pl.ds(buf * OUTW, OUTW)],
                              out_hbm.at[pl.ds(0, OUTW)],
                              osems[buf]).wait()

    def do_node(n, buf):
        ids = node_ids(n)
        for h in (0, 1):
            flat = ids[h] * C
            xs = [plsc.load_gather(coords_v, [flat + c]) for c in range(C)]
            rowstart = (iota + h * NLANE) * NLANE
            for c in range(C):
                plsc.store_scatter(y_v, [rowstart + c], xs[c])

        wait_gather(buf)
        @pl.when(n >= 2)
        def _():
            drain_out(buf)

        accs = []
        for cs, nch in PASSES:
            def k_body(k, acc):
                xk = y_v[pl.ds(k * NLANE, NLANE)]
                xb = [xk[c] for c in range(C)]
                new = list(acc)
                for ci in range(nch):
                    col = (cs + ci) * NLANE
                    wv = w_v[buf, k, pl.ds(col, NLANE)]
                    a = list(new[ci * NY:(ci + 1) * NY])
                    p = [wv * xb[c] for c in range(C)]
                    a[0] = a[0] + wv
                    for c in range(C):
                        a[1 + c] = a[1 + c] + p[c]
                    for t, (r, c) in enumerate(TRIL):
                        a[5 + t] = a[5 + t] + p[c] * xb[r]
                    new[ci * NY:(ci + 1) * NY] = a
                return tuple(new)

            zeros = tuple(jnp.zeros((NLANE,), jnp.float32)
                          for _ in range(nch * NY))
            accs.append((cs, nch,
                         lax.fori_loop(0, K, k_body, zeros)))

        @pl.when(n + 2 < count)
        def _():
            fire_gather(n + 2, buf)

        for cs, nch, acc in accs:
            for ci in range(nch):
                f0 = (cs + ci) * NLANE
                a = acc[ci * NY:(ci + 1) * NY]
                recip = 1.0 / (a[0] + EPS)
                m = [a[1 + c] * recip for c in range(C)]
                cov_base = buf * OUTW + (iota + f0) * len(TRIL)
                for t, (r, c) in enumerate(TRIL):
                    cov = a[5 + t] * recip - m[r] * m[c]
                    plsc.store_scatter(orow_v, [cov_base + t], cov)
                mean_base = buf * OUTW + COVW + (iota + f0) * C
                for c in range(C):
                    plsc.store_scatter(orow_v, [mean_base + c], m[c])

        pltpu.make_async_copy(orow_v.at[pl.ds(buf * OUTW, OUTW)],
                              out_hbm.at[pl.ds((base + n) * OUTW, OUTW)],
                              osems[buf]).start()

    fire_gather(0, 0)
    fire_gather(1, 1)

    def pair_body(i, carry):
        n0 = 2 * i

        @pl.when(n0 < count)
        def _():
            do_node(n0, 0)

        @pl.when(n0 + 1 < count)
        def _():
            do_node(n0 + 1, 1)

        return carry

    lax.fori_loop(0, (NBIG + 1) // 2, pair_body, 0)

    drain_out(0)
    drain_out(1)


@jax.jit
def kernel(coordinates, features, n_idxs):
    mesh = plsc.VectorSubcoreMesh(core_axis_name="c", subcore_axis_name="s")
    k = functools.partial(
        pl.kernel,
        out_type=jax.ShapeDtypeStruct((V * OUTW,), jnp.float32),
        mesh=mesh,
        compiler_params=pltpu.CompilerParams(needs_layout_passes=False),
        scratch_types=[
            pltpu.VMEM((V * C,), jnp.float32),
            pltpu.VMEM((NSTAGE * K,), jnp.int32),
            pltpu.VMEM((2, K, F), jnp.float32),
            pltpu.VMEM((K * NLANE,), jnp.float32),
            pltpu.VMEM((2 * OUTW,), jnp.float32),
            (pltpu.SemaphoreType.DMA, pltpu.SemaphoreType.DMA),
            (pltpu.SemaphoreType.DMA, pltpu.SemaphoreType.DMA),
        ],
    )(_body)
    return k(coordinates.reshape(-1), features,
             n_idxs.reshape(-1)).reshape(V, OUTW)
